# baseline (speedup 1.0000x reference)
import functools

import jax
import jax.numpy as jnp
from jax import lax
from jax.experimental import pallas as pl
from jax.experimental.pallas import tpu as pltpu
from jax.experimental.pallas import tpu_sc as plsc

G, M, DEG, H, OUT, CUTOFF = 250, 40, 16, 128, 12, 5.0
N = G * M
E = N * DEG
NC, NS = 1, 16
NW = NC * NS
NPW0 = N // NW
EPW = NPW0 * DEG
CPW = NPW0 * M
GB = 5
GSTEPS = G // GB


def _sc_count_body(ei_hbm, zeros_hbm, out_hbm, dst_v, c_v):
    wid = lax.axis_index("s")
    pltpu.sync_copy(ei_hbm.at[pl.ds(E + wid * EPW, EPW)], dst_v)
    pltpu.sync_copy(zeros_hbm, c_v)

    ones16 = jnp.ones((16,), jnp.float32)

    def node_body(i):
        dvec = dst_v[pl.ds(i * DEG, DEG)]
        lj = lax.rem(dvec, M)
        plsc.addupdate_scatter(c_v, [lj + i * M], ones16)

    plsc.parallel_loop(0, NPW0, unroll=8)(node_body)
    pltpu.sync_copy(c_v, out_hbm.at[pl.ds(wid * CPW, CPW)])


@functools.cache
def _sc_count():
    return pl.kernel(
        _sc_count_body,
        out_type=jax.ShapeDtypeStruct((N * M,), jnp.float32),
        mesh=plsc.VectorSubcoreMesh(core_axis_name="c", subcore_axis_name="s",
                                    num_cores=NC, num_subcores=NS),
        compiler_params=pltpu.CompilerParams(needs_layout_passes=False),
        scratch_types=[
            pltpu.VMEM((EPW,), jnp.int32),
            pltpu.VMEM((CPW,), jnp.float32),
        ],
    )

_DOT = dict(preferred_element_type=jnp.float32, precision=lax.Precision.HIGHEST)
_F32 = dict(preferred_element_type=jnp.float32)


def _split_bf16(v):
    hi = v.astype(jnp.bfloat16)
    lo = (v - hi.astype(jnp.float32)).astype(jnp.bfloat16)
    return hi, lo


def _dot_exact_bf16(a_bf, v_hi, v_lo, dims):
    return (lax.dot_general(a_bf, v_hi, dims, **_F32) +
            lax.dot_general(a_bf, v_lo, dims, **_F32))


BM = GB * M
HB = 2
STEP_G = HB * GB
TSTEPS = G // STEP_G


def _tc_main_body(z_ref, pt_ref, ptt_ref, cnt_ref, emb_ref, wmsg_ref, pw_ref,
                  pb_ref, eye_ref, sel_ref, out_ref, c_scr):
    embv = emb_ref[...]
    wmsg = wmsg_ref[...]
    pb = pb_ref[...]
    eyef = eye_ref[...]
    sel = sel_ref[...]
    kiota = lax.broadcasted_iota(jnp.int32, (100, BM), 0)
    emb_hi, emb_lo = _split_bf16(embv)
    sel_bf = sel.astype(jnp.bfloat16)

    @pl.when(pl.program_id(0) == 0)
    def _():
        c_scr[...] = jnp.zeros((HB, BM, BM), jnp.float32)

    for h in range(HB):
        for i in range(GB):
            r = h * BM + M * i
            c_scr[h, M * i:M * i + M, M * i:M * i + M] = \
                cnt_ref[0, r:r + M, :]

    for h in range(HB):
        cblk = c_scr[h]

        zall = z_ref[0, :, h * BM:(h + 1) * BM]
        onehot_t = (jnp.broadcast_to(zall, (100, BM)) == kiota
                    ).astype(jnp.bfloat16)
        x0 = _dot_exact_bf16(onehot_t, emb_hi, emb_lo,
                             (((0,), (0,)), ((), ())))

        pall = pt_ref[0, h * BM:(h + 1) * BM, :]
        prow = ptt_ref[0, :, h * BM:(h + 1) * BM]
        dx = pall[:, 0:1] - prow[0:1, :]
        dy = pall[:, 1:2] - prow[1:2, :]
        dz = pall[:, 2:3] - prow[2:3, :]
        d2 = jnp.minimum(dx * dx + dy * dy + dz * dz,
                         CUTOFF * CUTOFF)
        wm = (1.0 + d2 * (-0.098696012 + d2 * (0.00324695392 +
              d2 * (-4.27248512e-05 + d2 * (3.008280832e-07 +
              d2 * (-1.2982083584e-09 + d2 * 3.26407553024e-12))))))

        m1 = cblk * wm
        agg = lax.dot_general(m1, x0, (((0,), (0,)), ((), ())), **_DOT)
        pre = x0 + lax.dot_general(agg, wmsg,
                                   (((1,), (0,)), ((), ())), **_DOT)
        x = pre * jax.nn.sigmoid(pre)

        cblk_t = lax.dot_general(
            cblk.astype(jnp.bfloat16), eyef.astype(jnp.bfloat16),
            (((0,), (0,)), ((), ())), preferred_element_type=jnp.float32)
        adj_bf = ((cblk + cblk_t) > 0.0).astype(jnp.bfloat16)

        w_hi, w_lo = _split_bf16(x)
        ss = [_dot_exact_bf16(sel_bf, w_hi, w_lo, (((1,), (0,)), ((), ())))]
        for k in range(5):
            walk = _dot_exact_bf16(adj_bf, w_hi, w_lo,
                                   (((1,), (0,)), ((), ()))) * x
            w_hi, w_lo = _split_bf16(walk)
            ss.append(
                _dot_exact_bf16(sel_bf, w_hi, w_lo, (((1,), (0,)), ((), ()))))
        sflat = jnp.concatenate(ss, axis=1)
        o = pb + lax.dot_general(sflat, pw_ref[...],
                                 (((1,), (0,)), ((), ())), **_DOT)
        out_ref[0, h * GB:(h + 1) * GB, :] = o


def kernel(z, pos, batch, edge_index, emb, W_msg, proj_W, proj_b):
    zeros_c = jnp.zeros((CPW,), jnp.float32)
    cflat = _sc_count()(edge_index.reshape(2 * E), zeros_c)
    cnt3 = cflat.reshape(TSTEPS, STEP_G * M, M)

    z3 = z.reshape(TSTEPS, 1, HB * BM)
    pt3 = pos.reshape(TSTEPS, HB * BM, 3)
    ptt3 = pt3.transpose(0, 2, 1)
    pb2 = proj_b.reshape(1, OUT)
    eyef = jnp.eye(BM, dtype=jnp.float32)
    sel = (jnp.arange(BM, dtype=jnp.int32)[None, :] // M ==
           jnp.arange(GB, dtype=jnp.int32)[:, None]).astype(jnp.float32)

    out3 = pl.pallas_call(
        _tc_main_body,
        grid=(TSTEPS,),
        in_specs=[
            pl.BlockSpec((1, 1, HB * BM), lambda g: (g, 0, 0)),
            pl.BlockSpec((1, HB * BM, 3), lambda g: (g, 0, 0)),
            pl.BlockSpec((1, 3, HB * BM), lambda g: (g, 0, 0)),
            pl.BlockSpec((1, STEP_G * M, M), lambda g: (g, 0, 0)),
            pl.BlockSpec((100, H), lambda g: (0, 0)),
            pl.BlockSpec((H, H), lambda g: (0, 0)),
            pl.BlockSpec((6 * H, OUT), lambda g: (0, 0)),
            pl.BlockSpec((1, OUT), lambda g: (0, 0)),
            pl.BlockSpec((BM, BM), lambda g: (0, 0)),
            pl.BlockSpec((GB, BM), lambda g: (0, 0)),
        ],
        out_specs=pl.BlockSpec((1, STEP_G, OUT), lambda g: (g, 0, 0)),
        out_shape=jax.ShapeDtypeStruct((TSTEPS, STEP_G, OUT), jnp.float32),
        scratch_shapes=[pltpu.VMEM((HB, BM, BM), jnp.float32)],
    )(z3, pt3, ptt3, cnt3, emb, W_msg, proj_W, pb2, eyef, sel)
    return out3.reshape(G, OUT)


# speedup vs baseline: 1.2040x; 1.2040x over previous
import functools

import jax
import jax.numpy as jnp
from jax import lax
from jax.experimental import pallas as pl
from jax.experimental.pallas import tpu as pltpu
from jax.experimental.pallas import tpu_sc as plsc

G, M, DEG, H, OUT, CUTOFF = 250, 40, 16, 128, 12, 5.0
N = G * M
E = N * DEG
NC, NS = 2, 16
NW = NC * NS
NPW0 = 312
NLAST = N - (NW - 1) * NPW0
EMAX = NLAST * DEG
CMAX = NLAST * M
GB = 5
GSTEPS = G // GB


def _sc_count_body(ei_hbm, zeros_hbm, out_hbm, dst_v, c_v):
    wid = lax.axis_index("s") * NC + lax.axis_index("c")
    pltpu.sync_copy(ei_hbm.at[1, pl.ds(wid * NPW0 * DEG, EMAX)], dst_v)
    pltpu.sync_copy(zeros_hbm, c_v)

    ones16 = jnp.ones((16,), jnp.float32)

    def node_body(i, _):
        dvec = dst_v[pl.ds(i * DEG, DEG)]
        lj = lax.rem(dvec, M)
        plsc.addupdate_scatter(c_v, [lj + i * M], ones16)
        return 0

    lax.fori_loop(0, NPW0, node_body, 0, unroll=8)

    @pl.when(wid == NW - 1)
    def _():
        lax.fori_loop(NPW0, NLAST, node_body, 0, unroll=8)
    pltpu.sync_copy(c_v.at[pl.ds(0, NPW0 * M)],
                    out_hbm.at[pl.ds(wid * NPW0 * M, NPW0 * M)])

    @pl.when(wid == NW - 1)
    def _():
        pltpu.sync_copy(
            c_v.at[pl.ds(NPW0 * M, (NLAST - NPW0) * M)],
            out_hbm.at[pl.ds(NW * NPW0 * M, (NLAST - NPW0) * M)])


@functools.cache
def _sc_count():
    return pl.kernel(
        _sc_count_body,
        out_type=jax.ShapeDtypeStruct((N * M,), jnp.float32),
        mesh=plsc.VectorSubcoreMesh(core_axis_name="c", subcore_axis_name="s",
                                    num_cores=NC, num_subcores=NS),
        compiler_params=pltpu.CompilerParams(needs_layout_passes=False),
        scratch_types=[
            pltpu.VMEM((EMAX,), jnp.int32),
            pltpu.VMEM((CMAX,), jnp.float32),
        ],
    )

_DOT = dict(preferred_element_type=jnp.float32, precision=lax.Precision.HIGHEST)
_F32 = dict(preferred_element_type=jnp.float32)


def _split_bf16(v):
    hi = v.astype(jnp.bfloat16)
    lo = (v - hi.astype(jnp.float32)).astype(jnp.bfloat16)
    return hi, lo


def _dot_exact_bf16(a_bf, v_hi, v_lo, dims):
    return (lax.dot_general(a_bf, v_hi, dims, **_F32) +
            lax.dot_general(a_bf, v_lo, dims, **_F32))


BM = GB * M
HB = 2
STEP_G = HB * GB
TSTEPS = G // STEP_G


def _tc_main_body(z_ref, pt_ref, ptt_ref, cnt_ref, emb_ref, wmsg_ref, pw_ref,
                  pb_ref, eye_ref, sel_ref, out_ref, c_scr):
    embv = emb_ref[...]
    wmsg = wmsg_ref[...]
    pb = pb_ref[...]
    eyef = eye_ref[...]
    kiota = lax.broadcasted_iota(jnp.int32, (100, BM), 0)
    emb_hi, emb_lo = _split_bf16(embv)

    @pl.when(pl.program_id(0) == 0)
    def _():
        c_scr[...] = jnp.zeros((HB, BM, BM), jnp.float32)

    for h in range(HB):
        for i in range(GB):
            r = h * BM + M * i
            c_scr[h, M * i:M * i + M, M * i:M * i + M] = \
                cnt_ref[0, r:r + M, :]

    for h in range(HB):
        cblk = c_scr[h]

        zall = z_ref[0, :, h * BM:(h + 1) * BM]
        onehot_t = (jnp.broadcast_to(zall, (100, BM)) == kiota
                    ).astype(jnp.bfloat16)
        x0 = _dot_exact_bf16(onehot_t, emb_hi, emb_lo,
                             (((0,), (0,)), ((), ())))

        pall = pt_ref[0, h * BM:(h + 1) * BM, :]
        prow = ptt_ref[0, :, h * BM:(h + 1) * BM]
        dx = pall[:, 0:1] - prow[0:1, :]
        dy = pall[:, 1:2] - prow[1:2, :]
        dz = pall[:, 2:3] - prow[2:3, :]
        d2 = jnp.minimum(dx * dx + dy * dy + dz * dz,
                         CUTOFF * CUTOFF)
        wm = (1.0 + d2 * (-0.098696012 + d2 * (0.00324695392 +
              d2 * (-4.27248512e-05 + d2 * (3.008280832e-07 +
              d2 * (-1.2982083584e-09 + d2 * 3.26407553024e-12))))))

        m1 = cblk * wm
        agg = lax.dot_general(m1, x0, (((0,), (0,)), ((), ())), **_DOT)
        pre = x0 + lax.dot_general(agg, wmsg,
                                   (((1,), (0,)), ((), ())), **_DOT)
        x = pre * jax.nn.sigmoid(pre)

        cblk_t = lax.dot_general(
            cblk.astype(jnp.bfloat16), eyef.astype(jnp.bfloat16),
            (((0,), (0,)), ((), ())), preferred_element_type=jnp.float32)
        adj_bf = ((cblk + cblk_t) > 0.0).astype(jnp.bfloat16)

        w_hi, w_lo = _split_bf16(x)
        ss = [jnp.sum(x.reshape(GB, M, H), axis=1)]
        for k in range(5):
            walk = _dot_exact_bf16(adj_bf, w_hi, w_lo,
                                   (((1,), (0,)), ((), ()))) * x
            w_hi, w_lo = _split_bf16(walk)
            ss.append(jnp.sum(walk.reshape(GB, M, H), axis=1))
        sflat = jnp.concatenate(ss, axis=1)
        o = pb + lax.dot_general(sflat, pw_ref[...],
                                 (((1,), (0,)), ((), ())), **_DOT)
        out_ref[0, h * GB:(h + 1) * GB, :] = o


def kernel(z, pos, batch, edge_index, emb, W_msg, proj_W, proj_b):
    zeros_c = jnp.zeros((CMAX,), jnp.float32)
    cflat = _sc_count()(edge_index, zeros_c)
    cnt3 = cflat.reshape(TSTEPS, STEP_G * M, M)

    z3 = z.reshape(TSTEPS, 1, HB * BM)
    pt3 = pos.reshape(TSTEPS, HB * BM, 3)
    ptt3 = pt3.transpose(0, 2, 1)
    pb2 = proj_b.reshape(1, OUT)
    eyef = jnp.eye(BM, dtype=jnp.float32)
    sel = (jnp.arange(BM, dtype=jnp.int32)[None, :] // M ==
           jnp.arange(GB, dtype=jnp.int32)[:, None]).astype(jnp.float32)

    out3 = pl.pallas_call(
        _tc_main_body,
        grid=(TSTEPS,),
        in_specs=[
            pl.BlockSpec((1, 1, HB * BM), lambda g: (g, 0, 0)),
            pl.BlockSpec((1, HB * BM, 3), lambda g: (g, 0, 0)),
            pl.BlockSpec((1, 3, HB * BM), lambda g: (g, 0, 0)),
            pl.BlockSpec((1, STEP_G * M, M), lambda g: (g, 0, 0)),
            pl.BlockSpec((100, H), lambda g: (0, 0)),
            pl.BlockSpec((H, H), lambda g: (0, 0)),
            pl.BlockSpec((6 * H, OUT), lambda g: (0, 0)),
            pl.BlockSpec((1, OUT), lambda g: (0, 0)),
            pl.BlockSpec((BM, BM), lambda g: (0, 0)),
            pl.BlockSpec((GB, BM), lambda g: (0, 0)),
        ],
        out_specs=pl.BlockSpec((1, STEP_G, OUT), lambda g: (g, 0, 0)),
        out_shape=jax.ShapeDtypeStruct((TSTEPS, STEP_G, OUT), jnp.float32),
        scratch_shapes=[pltpu.VMEM((HB, BM, BM), jnp.float32)],
    )(z3, pt3, ptt3, cnt3, emb, W_msg, proj_W, pb2, eyef, sel)
    return out3.reshape(G, OUT)
